# baseline (speedup 1.0000x reference)
import functools

import jax
import jax.numpy as jnp
from jax import lax
from jax.experimental import pallas as pl
from jax.experimental.pallas import tpu as pltpu
from jax.experimental.pallas import tpu_sc as plsc

B = 16384
PN = 26
V = 100
D = 128
OUT = 1024
IN = PN * D

NC = 2
NS = 16
NW = NC * NS

NCH = 8
CH = B // NCH
BCW = CH // NW


def _sc_gather_chunk(xt_c: jax.Array, emb_flat: jax.Array) -> jax.Array:
    mesh = plsc.VectorSubcoreMesh(core_axis_name="c", subcore_axis_name="s")

    @functools.partial(
        pl.kernel,
        out_type=jax.ShapeDtypeStruct((CH, IN), jnp.float32),
        mesh=mesh,
        scratch_types=[
            pltpu.VMEM((PN, 2 * BCW), jnp.int32),
            *[pltpu.VMEM((BCW, D), jnp.float32) for _ in range(4)],
            *[pltpu.SemaphoreType.DMA for _ in range(8)],
        ],
    )
    def k(xt_hbm, emb_hbm, pe_hbm, idx2,
          b0, b1, b2, b3, g0, g1, g2, g3, w0, w1, w2, w3):
        bufs = (b0, b1, b2, b3)
        gs = (g0, g1, g2, g3)
        ws = (w0, w1, w2, w3)
        wid = lax.axis_index("s") * NC + lax.axis_index("c")
        base = wid * BCW
        half = (wid % 2) * BCW

        pltpu.sync_copy(xt_hbm.at[:, pl.ds((wid // 2) * 2 * BCW, 2 * BCW)],
                        idx2)

        def gather_start(i, p):
            pltpu.async_copy(emb_hbm.at[idx2.at[i, pl.ds(half, BCW)]],
                             bufs[p], gs[p])

        def gather_wait(p):
            pltpu.make_async_copy(
                emb_hbm.at[pl.ds(0, BCW)], bufs[p], gs[p]).wait()

        def write_start(i, p):
            pltpu.async_copy(
                bufs[p],
                pe_hbm.at[pl.ds(base, BCW), pl.ds(i * D, D)],
                ws[p])

        def write_wait(p):
            pltpu.make_async_copy(
                bufs[p],
                pe_hbm.at[pl.ds(0, BCW), pl.ds(0, D)],
                ws[p]).wait()

        gather_start(0, 0)
        gather_start(1, 1)
        for t in (0, 1):
            gather_wait(t)
            write_start(t, t)
            gather_start(t + 2, (t + 2) % 4)

        def body(j, carry):
            for p in range(4):
                t = 4 * j + 2 + p
                q = (p + 2) % 4
                gather_wait(q)
                write_start(t, q)
                write_wait(p)
                gather_start(t + 2, p)
            return carry

        lax.fori_loop(0, 5, body, 0)

        for t in (22, 23):
            q = t % 4
            gather_wait(q)
            write_start(t, q)
            write_wait((q + 2) % 4)
            gather_start(t + 2, (q + 2) % 4)
        for t in (24, 25):
            q = t % 4
            gather_wait(q)
            write_start(t, q)
        for p in range(4):
            write_wait(p)

    return k(xt_c, emb_flat)


def _tc_heads_chunk(c, prev, pe_c, wd_t, wp_t, bd, bp):
    BB = 512
    nb = CH // BB

    def mm(*refs):
        pe_ref, wd_ref, wp_ref, bd_ref, bp_ref = refs[-8:-3]
        dg_ref, pc_ref, dg2_ref = refs[-3:]
        a = pe_ref[...].astype(jnp.bfloat16)
        dg = (
            jnp.dot(a, wd_ref[...], preferred_element_type=jnp.float32)
            + bd_ref[...]
        )
        dg_ref[...] = dg
        dg2_ref[...] = dg
        pc_ref[...] = (
            jnp.dot(a, wp_ref[...], preferred_element_type=jnp.float32)
            + bp_ref[...]
        )

    alias_specs = [] if c == 0 else [pl.BlockSpec(memory_space=pl.ANY)] * 3
    alias_args = () if c == 0 else tuple(prev)
    out_spec = pl.BlockSpec(
        (BB, OUT), lambda b, _c=c, _nb=nb: (b + _c * _nb, 0))
    return pl.pallas_call(
        mm,
        grid=(nb,),
        in_specs=alias_specs + [
            pl.BlockSpec((BB, IN), lambda b: (b, 0)),
            pl.BlockSpec((IN, OUT), lambda b: (0, 0)),
            pl.BlockSpec((IN, OUT), lambda b: (0, 0)),
            pl.BlockSpec((1, OUT), lambda b: (0, 0)),
            pl.BlockSpec((1, OUT), lambda b: (0, 0)),
        ],
        out_specs=[out_spec, out_spec, out_spec],
        out_shape=[
            jax.ShapeDtypeStruct((B, OUT), jnp.float32),
            jax.ShapeDtypeStruct((B, OUT), jnp.float32),
            jax.ShapeDtypeStruct((B, OUT), jnp.float32),
        ],
        input_output_aliases={} if c == 0 else {0: 0, 1: 1, 2: 2},
    )(*alias_args, pe_c, wd_t, wp_t, bd, bp)


def kernel(x, emb_tables, W_diag, b_diag, W_proc, b_proc):
    xt_off = (
        x.astype(jnp.int32) + V * jnp.arange(PN, dtype=jnp.int32)[None, :]
    ).T
    emb_flat = emb_tables.reshape(PN * V, D)

    wd_t = W_diag.T.astype(jnp.bfloat16)
    wp_t = W_proc.T.astype(jnp.bfloat16)
    bd = b_diag.reshape(1, OUT)
    bp = b_proc.reshape(1, OUT)

    prev = None
    for c in range(NCH):
        pe_c = _sc_gather_chunk(
            xt_off[:, c * CH:(c + 1) * CH], emb_flat)
        prev = _tc_heads_chunk(c, prev, pe_c, wd_t, wp_t, bd, bp)

    diag, proc, diag2 = prev
    return (diag, proc, diag2)


# speedup vs baseline: 1.0242x; 1.0242x over previous
import functools

import jax
import jax.numpy as jnp
from jax import lax
from jax.experimental import pallas as pl
from jax.experimental.pallas import tpu as pltpu
from jax.experimental.pallas import tpu_sc as plsc

B = 16384
PN = 26
V = 100
D = 128
OUT = 1024
IN = PN * D

NC = 2
NS = 16
NW = NC * NS

NCH = 4
CH = B // NCH
BCW = CH // NW


def _sc_gather_chunk(xt_c: jax.Array, emb_flat: jax.Array) -> jax.Array:
    mesh = plsc.VectorSubcoreMesh(core_axis_name="c", subcore_axis_name="s")

    @functools.partial(
        pl.kernel,
        out_type=jax.ShapeDtypeStruct((CH, IN), jnp.float32),
        mesh=mesh,
        scratch_types=[
            pltpu.VMEM((PN, BCW), jnp.int32),
            *[pltpu.VMEM((BCW, D), jnp.float32) for _ in range(4)],
            *[pltpu.SemaphoreType.DMA for _ in range(8)],
        ],
    )
    def k(xt_hbm, emb_hbm, pe_hbm, idx2,
          b0, b1, b2, b3, g0, g1, g2, g3, w0, w1, w2, w3):
        bufs = (b0, b1, b2, b3)
        gs = (g0, g1, g2, g3)
        ws = (w0, w1, w2, w3)
        wid = lax.axis_index("s") * NC + lax.axis_index("c")
        base = wid * BCW

        pltpu.sync_copy(xt_hbm.at[:, pl.ds(base, BCW)], idx2)

        def gather_start(i, p):
            pltpu.async_copy(emb_hbm.at[idx2.at[i]], bufs[p], gs[p])

        def gather_wait(p):
            pltpu.make_async_copy(
                emb_hbm.at[pl.ds(0, BCW)], bufs[p], gs[p]).wait()

        def write_start(i, p):
            pltpu.async_copy(
                bufs[p],
                pe_hbm.at[pl.ds(base, BCW), pl.ds(i * D, D)],
                ws[p])

        def write_wait(p):
            pltpu.make_async_copy(
                bufs[p],
                pe_hbm.at[pl.ds(0, BCW), pl.ds(0, D)],
                ws[p]).wait()

        gather_start(0, 0)
        gather_start(1, 1)
        for t in (0, 1):
            gather_wait(t)
            write_start(t, t)
            gather_start(t + 2, (t + 2) % 4)

        def body(j, carry):
            for p in range(4):
                t = 4 * j + 2 + p
                q = (p + 2) % 4
                gather_wait(q)
                write_start(t, q)
                write_wait(p)
                gather_start(t + 2, p)
            return carry

        lax.fori_loop(0, 5, body, 0)

        for t in (22, 23):
            q = t % 4
            gather_wait(q)
            write_start(t, q)
            write_wait((q + 2) % 4)
            gather_start(t + 2, (q + 2) % 4)
        for t in (24, 25):
            q = t % 4
            gather_wait(q)
            write_start(t, q)
        for p in range(4):
            write_wait(p)

    return k(xt_c, emb_flat)


def _tc_heads_chunk(c, prev, pe_c, wd_t, wp_t, bd, bp):
    BB = 512
    nb = CH // BB

    def mm(*refs):
        pe_ref, wd_ref, wp_ref, bd_ref, bp_ref = refs[-8:-3]
        dg_ref, pc_ref, dg2_ref = refs[-3:]
        a = pe_ref[...].astype(jnp.bfloat16)
        dg = (
            jnp.dot(a, wd_ref[...], preferred_element_type=jnp.float32)
            + bd_ref[...]
        )
        dg_ref[...] = dg
        dg2_ref[...] = dg
        pc_ref[...] = (
            jnp.dot(a, wp_ref[...], preferred_element_type=jnp.float32)
            + bp_ref[...]
        )

    alias_specs = [] if c == 0 else [pl.BlockSpec(memory_space=pl.ANY)] * 3
    alias_args = () if c == 0 else tuple(prev)
    out_spec = pl.BlockSpec(
        (BB, OUT), lambda b, _c=c, _nb=nb: (b + _c * _nb, 0))
    return pl.pallas_call(
        mm,
        grid=(nb,),
        in_specs=alias_specs + [
            pl.BlockSpec((BB, IN), lambda b: (b, 0)),
            pl.BlockSpec((IN, OUT), lambda b: (0, 0)),
            pl.BlockSpec((IN, OUT), lambda b: (0, 0)),
            pl.BlockSpec((1, OUT), lambda b: (0, 0)),
            pl.BlockSpec((1, OUT), lambda b: (0, 0)),
        ],
        out_specs=[out_spec, out_spec, out_spec],
        out_shape=[
            jax.ShapeDtypeStruct((B, OUT), jnp.float32),
            jax.ShapeDtypeStruct((B, OUT), jnp.float32),
            jax.ShapeDtypeStruct((B, OUT), jnp.float32),
        ],
        input_output_aliases={} if c == 0 else {0: 0, 1: 1, 2: 2},
    )(*alias_args, pe_c, wd_t, wp_t, bd, bp)


def kernel(x, emb_tables, W_diag, b_diag, W_proc, b_proc):
    xt_off = (
        x.astype(jnp.int32) + V * jnp.arange(PN, dtype=jnp.int32)[None, :]
    ).T
    emb_flat = emb_tables.reshape(PN * V, D)

    wd_t = W_diag.T.astype(jnp.bfloat16)
    wp_t = W_proc.T.astype(jnp.bfloat16)
    bd = b_diag.reshape(1, OUT)
    bp = b_proc.reshape(1, OUT)

    prev = None
    for c in range(NCH):
        pe_c = _sc_gather_chunk(
            xt_off[:, c * CH:(c + 1) * CH], emb_flat)
        prev = _tc_heads_chunk(c, prev, pe_c, wd_t, wp_t, bd, bp)

    diag, proc, diag2 = prev
    return (diag, proc, diag2)


# speedup vs baseline: 1.0904x; 1.0646x over previous
import functools

import numpy as np
import jax
import jax.numpy as jnp
from jax import lax
from jax.experimental import pallas as pl
from jax.experimental.pallas import tpu as pltpu
from jax.experimental.pallas import tpu_sc as plsc

B = 16384
PN = 26
V = 100
D = 128
OUT = 1024
IN = PN * D
INW = IN // 2

NC = 2
NS = 16
NW = NC * NS

NCH = 4
CH = B // NCH
BCW = CH // NW
L = 16


def _sc_gather_chunk(xt_c: jax.Array, emb_flat: jax.Array) -> jax.Array:
    mesh = plsc.VectorSubcoreMesh(core_axis_name="c", subcore_axis_name="s")

    @functools.partial(
        pl.kernel,
        out_type=jax.ShapeDtypeStruct((CH, INW), jnp.int32),
        mesh=mesh,
        scratch_types=[
            pltpu.VMEM((PN, BCW), jnp.int32),
            *[pltpu.VMEM((BCW, D), jnp.float32) for _ in range(4)],
            *[pltpu.VMEM((BCW, D), jnp.int32) for _ in range(2)],
            *[pltpu.SemaphoreType.DMA for _ in range(6)],
        ],
    )
    def k(xt_hbm, emb_hbm, pe_hbm, idx2,
          b0, b1, b2, b3, o0, o1, g0, g1, g2, g3, w0, w1):
        bufs = (b0, b1, b2, b3)
        obufs = (o0, o1)
        gs = (g0, g1, g2, g3)
        ws = (w0, w1)
        wid = lax.axis_index("s") * NC + lax.axis_index("c")
        base = wid * BCW

        pltpu.sync_copy(xt_hbm.at[:, pl.ds(base, BCW)], idx2)

        def gather_start(i, p):
            pltpu.async_copy(emb_hbm.at[idx2.at[i]], bufs[p], gs[p])

        def gather_wait(p):
            pltpu.make_async_copy(
                emb_hbm.at[pl.ds(0, BCW)], bufs[p], gs[p]).wait()

        def write_start(P, ob):
            pltpu.async_copy(
                obufs[ob],
                pe_hbm.at[pl.ds(base, BCW), pl.ds(P * D, D)],
                ws[ob])

        def write_wait(ob):
            pltpu.make_async_copy(
                obufs[ob],
                pe_hbm.at[pl.ds(0, BCW), pl.ds(0, D)],
                ws[ob]).wait()

        def rtne(v):
            xb = lax.bitcast_convert_type(v, jnp.int32)
            lsb = lax.shift_right_logical(xb, 16) & 1
            return lax.shift_right_logical(xb + 32767 + lsb, 16)

        def convert(q, ob, h):
            def row(r, carry):
                for g in range(4):
                    a = rtne(bufs[q][r, pl.ds(32 * g, L)])
                    b = rtne(bufs[q][r, pl.ds(32 * g + L, L)])
                    obufs[ob][r, pl.ds(h * 64 + L * g, L)] = (
                        a | lax.shift_left(b, 16))
                return carry
            lax.fori_loop(0, BCW, row, 0)

        for p in range(4):
            gather_start(p, p)
        for t in range(4):
            q, h, ob = t % 4, t % 2, (t // 2) % 2
            gather_wait(q)
            convert(q, ob, h)
            gather_start(t + 4, q)
            if h == 1:
                write_start(t // 2, ob)

        def body(j, carry):
            for p in range(4):
                t = 4 * j + p
                q, h, ob = p % 4, p % 2, (p // 2) % 2
                P = 2 * j + p // 2
                gather_wait(q)
                if h == 0:
                    write_wait(ob)
                convert(q, ob, h)
                gather_start(t + 4, q)
                if h == 1:
                    write_start(P, ob)
            return carry

        lax.fori_loop(1, 5, body, 0)

        for t in (20, 21, 22, 23):
            q, h, ob = t % 4, t % 2, ((t // 2) % 2)
            P = t // 2
            gather_wait(q)
            if h == 0:
                write_wait(ob)
            convert(q, ob, h)
            if t < 22:
                gather_start(t + 4, q)
            if h == 1:
                write_start(P, ob)

        gather_wait(0)
        write_wait(0)
        convert(0, 0, 0)
        gather_wait(1)
        convert(1, 0, 1)
        write_start(12, 0)

        write_wait(0)
        write_wait(1)

    return k(xt_c, emb_flat)


def _w_perm() -> np.ndarray:
    perm = np.zeros(IN, np.int64)
    for p in range(INW):
        i, j = p // 64, p % 64
        perm[p] = i * 128 + 32 * (j // 16) + (j % 16)
        perm[INW + p] = perm[p] + 16
    return perm


def _tc_heads_chunk(c, prev, pe_c, wd_p, wp_p, bd, bp):
    BB = 512
    nb = CH // BB

    def mm(*refs):
        pe_ref, wd_ref, wp_ref, bd_ref, bp_ref = refs[-8:-3]
        dg_ref, pc_ref, dg2_ref = refs[-3:]
        x = pe_ref[...]
        mask = jnp.full(x.shape, -65536, jnp.int32)
        lo = lax.bitcast_convert_type(
            lax.shift_left(x, 16), jnp.float32).astype(jnp.bfloat16)
        hi = lax.bitcast_convert_type(
            lax.bitwise_and(x, mask), jnp.float32).astype(jnp.bfloat16)
        a = jnp.concatenate([lo, hi], axis=1)
        dg = (
            jnp.dot(a, wd_ref[...], preferred_element_type=jnp.float32)
            + bd_ref[...]
        )
        dg_ref[...] = dg
        dg2_ref[...] = dg
        pc_ref[...] = (
            jnp.dot(a, wp_ref[...], preferred_element_type=jnp.float32)
            + bp_ref[...]
        )

    alias_specs = [] if c == 0 else [pl.BlockSpec(memory_space=pl.ANY)] * 3
    alias_args = () if c == 0 else tuple(prev)
    out_spec = pl.BlockSpec(
        (BB, OUT), lambda b, _c=c, _nb=nb: (b + _c * _nb, 0))
    return pl.pallas_call(
        mm,
        grid=(nb,),
        in_specs=alias_specs + [
            pl.BlockSpec((BB, INW), lambda b: (b, 0)),
            pl.BlockSpec((IN, OUT), lambda b: (0, 0)),
            pl.BlockSpec((IN, OUT), lambda b: (0, 0)),
            pl.BlockSpec((1, OUT), lambda b: (0, 0)),
            pl.BlockSpec((1, OUT), lambda b: (0, 0)),
        ],
        out_specs=[out_spec, out_spec, out_spec],
        out_shape=[
            jax.ShapeDtypeStruct((B, OUT), jnp.float32),
            jax.ShapeDtypeStruct((B, OUT), jnp.float32),
            jax.ShapeDtypeStruct((B, OUT), jnp.float32),
        ],
        input_output_aliases={} if c == 0 else {0: 0, 1: 1, 2: 2},
    )(*alias_args, pe_c, wd_p, wp_p, bd, bp)


def kernel(x, emb_tables, W_diag, b_diag, W_proc, b_proc):
    xt_off = (
        x.astype(jnp.int32) + V * jnp.arange(PN, dtype=jnp.int32)[None, :]
    ).T
    emb_flat = emb_tables.reshape(PN * V, D)

    perm = jnp.asarray(_w_perm())
    wd_p = W_diag.T.astype(jnp.bfloat16)[perm]
    wp_p = W_proc.T.astype(jnp.bfloat16)[perm]
    bd = b_diag.reshape(1, OUT)
    bp = b_proc.reshape(1, OUT)

    prev = None
    for c in range(NCH):
        pe_c = _sc_gather_chunk(
            xt_off[:, c * CH:(c + 1) * CH], emb_flat)
        prev = _tc_heads_chunk(c, prev, pe_c, wd_p, wp_p, bd, bp)

    diag, proc, diag2 = prev
    return (diag, proc, diag2)


# speedup vs baseline: 1.1614x; 1.0651x over previous
import functools

import jax
import jax.numpy as jnp
from jax import lax
from jax.experimental import pallas as pl
from jax.experimental.pallas import tpu as pltpu
from jax.experimental.pallas import tpu_sc as plsc

B = 16384
PN = 26
V = 100
D = 128
OUT = 1024
IN = PN * D
INW = IN // 2

NC = 2
NS = 16
NW = NC * NS

NCH = 4
CH = B // NCH
BCW = CH // NW
L = 16


def _sc_gather_chunk(xt_c: jax.Array, emb_flat: jax.Array) -> jax.Array:
    mesh = plsc.VectorSubcoreMesh(core_axis_name="c", subcore_axis_name="s")

    @functools.partial(
        pl.kernel,
        out_type=jax.ShapeDtypeStruct((CH, INW), jnp.int32),
        mesh=mesh,
        scratch_types=[
            pltpu.VMEM((PN, BCW), jnp.int32),
            *[pltpu.VMEM((BCW, D), jnp.float32) for _ in range(4)],
            *[pltpu.VMEM((BCW, D), jnp.int32) for _ in range(2)],
            *[pltpu.SemaphoreType.DMA for _ in range(6)],
        ],
    )
    def k(xt_hbm, emb_hbm, pe_hbm, idx2,
          b0, b1, b2, b3, o0, o1, g0, g1, g2, g3, w0, w1):
        bufs = (b0, b1, b2, b3)
        obufs = (o0, o1)
        gs = (g0, g1, g2, g3)
        ws = (w0, w1)
        wid = lax.axis_index("s") * NC + lax.axis_index("c")
        base = wid * BCW

        pltpu.sync_copy(xt_hbm.at[:, pl.ds(base, BCW)], idx2)

        def gather_start(i, p):
            pltpu.async_copy(emb_hbm.at[idx2.at[i]], bufs[p], gs[p])

        def gather_wait(p):
            pltpu.make_async_copy(
                emb_hbm.at[pl.ds(0, BCW)], bufs[p], gs[p]).wait()

        def write_start(P, ob):
            pltpu.async_copy(
                obufs[ob],
                pe_hbm.at[pl.ds(base, BCW), pl.ds(P * D, D)],
                ws[ob])

        def write_wait(ob):
            pltpu.make_async_copy(
                obufs[ob],
                pe_hbm.at[pl.ds(0, BCW), pl.ds(0, D)],
                ws[ob]).wait()

        def rtne(v):
            xb = lax.bitcast_convert_type(v, jnp.int32)
            lsb = lax.shift_right_logical(xb, 16) & 1
            return lax.shift_right_logical(xb + 32767 + lsb, 16)

        def convert(qa, qb, ob):
            def row(r, carry):
                for g in range(8):
                    a = rtne(bufs[qa][r, pl.ds(L * g, L)])
                    b = rtne(bufs[qb][r, pl.ds(L * g, L)])
                    obufs[ob][r, pl.ds(L * g, L)] = (
                        a | lax.shift_left(b, 16))
                return carry
            lax.fori_loop(0, BCW, row, 0)

        NP = PN // 2

        gather_start(0, 0)
        gather_start(13, 1)
        gather_start(1, 2)
        gather_start(14, 3)
        for P in (0, 1):
            qa, qb, ob = (2 * P) % 4, (2 * P + 1) % 4, P % 2
            gather_wait(qa)
            gather_wait(qb)
            convert(qa, qb, ob)
            gather_start(P + 2, qa)
            gather_start(P + 15, qb)
            write_start(P, ob)

        def body(j, carry):
            for par in range(2):
                P = 2 * j + par
                qa, qb, ob = 2 * par, 2 * par + 1, par
                gather_wait(qa)
                gather_wait(qb)
                write_wait(ob)
                convert(qa, qb, ob)
                gather_start(P + 2, qa)
                gather_start(P + 15, qb)
                write_start(P, ob)
            return carry

        lax.fori_loop(1, 5, body, 0)

        gather_wait(0)
        gather_wait(1)
        write_wait(0)
        convert(0, 1, 0)
        gather_start(12, 0)
        gather_start(25, 1)
        write_start(10, 0)

        gather_wait(2)
        gather_wait(3)
        write_wait(1)
        convert(2, 3, 1)
        write_start(11, 1)

        gather_wait(0)
        gather_wait(1)
        write_wait(0)
        convert(0, 1, 0)
        write_start(12, 0)

        write_wait(0)
        write_wait(1)

    return k(xt_c, emb_flat)


def _tc_heads_chunk(c, prev, pe_c, wd_p, wp_p, bd, bp):
    BB = 512
    nb = CH // BB

    def mm(*refs):
        pe_ref, wd_ref, wp_ref, bd_ref, bp_ref = refs[-8:-3]
        dg_ref, pc_ref, dg2_ref = refs[-3:]
        x = pe_ref[...]
        mask = jnp.full(x.shape, -65536, jnp.int32)
        lo = lax.bitcast_convert_type(
            lax.shift_left(x, 16), jnp.float32).astype(jnp.bfloat16)
        hi = lax.bitcast_convert_type(
            lax.bitwise_and(x, mask), jnp.float32).astype(jnp.bfloat16)
        a = jnp.concatenate([lo, hi], axis=1)
        dg = (
            jnp.dot(a, wd_ref[...], preferred_element_type=jnp.float32)
            + bd_ref[...]
        )
        dg_ref[...] = dg
        dg2_ref[...] = dg
        pc_ref[...] = (
            jnp.dot(a, wp_ref[...], preferred_element_type=jnp.float32)
            + bp_ref[...]
        )

    alias_specs = [] if c == 0 else [pl.BlockSpec(memory_space=pl.ANY)] * 3
    alias_args = () if c == 0 else tuple(prev)
    out_spec = pl.BlockSpec(
        (BB, OUT), lambda b, _c=c, _nb=nb: (b + _c * _nb, 0))
    return pl.pallas_call(
        mm,
        grid=(nb,),
        in_specs=alias_specs + [
            pl.BlockSpec((BB, INW), lambda b: (b, 0)),
            pl.BlockSpec((IN, OUT), lambda b: (0, 0)),
            pl.BlockSpec((IN, OUT), lambda b: (0, 0)),
            pl.BlockSpec((1, OUT), lambda b: (0, 0)),
            pl.BlockSpec((1, OUT), lambda b: (0, 0)),
        ],
        out_specs=[out_spec, out_spec, out_spec],
        out_shape=[
            jax.ShapeDtypeStruct((B, OUT), jnp.float32),
            jax.ShapeDtypeStruct((B, OUT), jnp.float32),
            jax.ShapeDtypeStruct((B, OUT), jnp.float32),
        ],
        input_output_aliases={} if c == 0 else {0: 0, 1: 1, 2: 2},
    )(*alias_args, pe_c, wd_p, wp_p, bd, bp)


def kernel(x, emb_tables, W_diag, b_diag, W_proc, b_proc):
    xt_off = (
        x.astype(jnp.int32) + V * jnp.arange(PN, dtype=jnp.int32)[None, :]
    ).T
    emb_flat = emb_tables.reshape(PN * V, D)

    wd_p = W_diag.T.astype(jnp.bfloat16)
    wp_p = W_proc.T.astype(jnp.bfloat16)
    bd = b_diag.reshape(1, OUT)
    bp = b_proc.reshape(1, OUT)

    prev = None
    for c in range(NCH):
        pe_c = _sc_gather_chunk(
            xt_off[:, c * CH:(c + 1) * CH], emb_flat)
        prev = _tc_heads_chunk(c, prev, pe_c, wd_p, wp_p, bd, bp)

    diag, proc, diag2 = prev
    return (diag, proc, diag2)
